# baseline (speedup 1.0000x reference)
import jax
import jax.numpy as jnp
from jax import lax
from jax.experimental import pallas as pl
from jax.experimental.pallas import tpu as pltpu
from jax.experimental.pallas import tpu_sc as plsc

_B = 4
_SEG = 8192
_NS = 2049
_K = 16
_IN_C = 64
_OUT_C = 128
_QR = 17
_NSP = _QR * 128
_N = _B * _SEG
_SR = _SEG // 128


def _fps_body(x_ref, y_ref, z_ref, q_ref):
    shape = (_B, _SR, 128)
    lin = (lax.broadcasted_iota(jnp.int32, shape, 1) * 128
           + lax.broadcasted_iota(jnp.int32, shape, 2))
    q_ref[...] = jnp.zeros((_NSP, 16), jnp.float32)

    def store_q(i, qx, qy, qz):
        for s in range(_B):
            q_ref[pl.ds(i, 1), s:s + 1] = qx[s]
            q_ref[pl.ds(i, 1), _B + s:_B + s + 1] = qy[s]
            q_ref[pl.ds(i, 1), 2 * _B + s:2 * _B + s + 1] = qz[s]

    def extract(wm):
        qx = jnp.sum(jnp.where(wm, x_ref[...], 0.0), axis=(1, 2), keepdims=True)
        qy = jnp.sum(jnp.where(wm, y_ref[...], 0.0), axis=(1, 2), keepdims=True)
        qz = jnp.sum(jnp.where(wm, z_ref[...], 0.0), axis=(1, 2), keepdims=True)
        return qx, qy, qz

    def body(i, carry):
        dists, qx, qy, qz = carry
        dx = x_ref[...] - qx
        dy = y_ref[...] - qy
        dz = z_ref[...] - qz
        d = (dx * dx + dz * dz) + dy * dy
        dists = jnp.minimum(dists, d)
        mx = jnp.max(dists, axis=(1, 2), keepdims=True)
        cand = jnp.where(dists == mx, lin, _SEG)
        j = jnp.min(cand, axis=(1, 2), keepdims=True)
        qx2, qy2, qz2 = extract(cand == j)
        store_q(i + 1, qx2, qy2, qz2)
        return dists, qx2, qy2, qz2

    qx0, qy0, qz0 = extract(lin == 0)
    store_q(0, qx0, qy0, qz0)
    init = (jnp.full(shape, jnp.inf, jnp.float32), qx0, qy0, qz0)
    lax.fori_loop(0, _NS - 1, body, init)


def _run_fps(px, py, pz):
    return pl.pallas_call(
        _fps_body,
        out_shape=jax.ShapeDtypeStruct((_NSP, 16), jnp.float32),
    )(px.reshape(_B, _SR, 128), py.reshape(_B, _SR, 128),
      pz.reshape(_B, _SR, 128))


def _z_body(f_ref, g_ref, b_ref, w_ref, z_ref):
    x = f_ref[...]
    mu = jnp.mean(x, axis=1, keepdims=True)
    xc = x - mu
    var = jnp.mean(xc * xc, axis=1, keepdims=True)
    gn = xc / jnp.sqrt(var + 1e-5) * g_ref[...] + b_ref[...]
    z_ref[...] = jnp.dot(gn, w_ref[...],
                         preferred_element_type=jnp.float32,
                         precision=lax.Precision.HIGHEST)


def _run_z(feats, gamma, beta, wT):
    rows = min(2048, _N)
    return pl.pallas_call(
        _z_body,
        grid=(_N // rows,),
        in_specs=[
            pl.BlockSpec((rows, _IN_C), lambda i: (i, 0)),
            pl.BlockSpec((1, _IN_C), lambda i: (0, 0)),
            pl.BlockSpec((1, _IN_C), lambda i: (0, 0)),
            pl.BlockSpec((_IN_C, _OUT_C), lambda i: (0, 0)),
        ],
        out_specs=pl.BlockSpec((rows, _OUT_C), lambda i: (i, 0)),
        out_shape=jax.ShapeDtypeStruct((_N, _OUT_C), jnp.float32),
    )(feats, gamma.reshape(1, _IN_C), beta.reshape(1, _IN_C), wT)


def _knn_body(qx_ref, qy_ref, qz_ref, px_ref, py_ref, pz_ref, idx_ref, d_ref):
    s = pl.program_id(0)
    qx = jnp.swapaxes(qx_ref[0, 0], 0, 1)
    qy = jnp.swapaxes(qy_ref[0, 0], 0, 1)
    qz = jnp.swapaxes(qz_ref[0, 0], 0, 1)
    dx = qx - px_ref[0]
    dy = qy - py_ref[0]
    dz = qz - pz_ref[0]
    d_ref[...] = dx * dx + dy * dy + dz * dz
    iota = lax.broadcasted_iota(jnp.int32, (128, _SEG), 1)
    for k in range(_K):
        dmat = d_ref[...]
        mn = jnp.min(dmat, axis=1, keepdims=True)
        cand = jnp.where(dmat == mn, iota, _SEG)
        j = jnp.min(cand, axis=1, keepdims=True)
        idx_ref[0, :, k:k + 1] = j + s * _SEG
        d_ref[...] = jnp.where(iota == j, jnp.inf, dmat)


def _run_knn(qx, qy, qz, px, py, pz):
    qspec = pl.BlockSpec((1, 1, 1, 128), lambda s, qb: (s, qb, 0, 0))
    pspec = pl.BlockSpec((1, 1, _SEG), lambda s, qb: (s, 0, 0))
    return pl.pallas_call(
        _knn_body,
        grid=(_B, _QR),
        in_specs=[qspec, qspec, qspec, pspec, pspec, pspec],
        out_specs=pl.BlockSpec((1, 128, _K), lambda s, qb: (s, qb, 0)),
        out_shape=jax.ShapeDtypeStruct((_B, _NSP, _K), jnp.int32),
        scratch_shapes=[pltpu.VMEM((128, _SEG), jnp.float32)],
    )(qx.reshape(_B, _QR, 1, 128), qy.reshape(_B, _QR, 1, 128),
      qz.reshape(_B, _QR, 1, 128),
      px.reshape(_B, 1, _SEG), py.reshape(_B, 1, _SEG),
      pz.reshape(_B, 1, _SEG))


_SC_ROWS = 8


def _run_gather_max(z, idx):
    m = _B * _NSP
    mesh = plsc.VectorSubcoreMesh(core_axis_name="core",
                                  subcore_axis_name="subcore")

    @pl.kernel(out_type=jax.ShapeDtypeStruct((m, _OUT_C), jnp.float32),
               mesh=mesh,
               scratch_types=[pltpu.VMEM((_K, _OUT_C), jnp.float32)])
    def gather_kernel(z_hbm, i_hbm, o_hbm, scratch):
        def body(i_vmem, o_vmem):
            @pl.loop(0, _SC_ROWS)
            def _(r):
                pltpu.sync_copy(z_hbm.at[i_vmem.at[r]], scratch)
                for c in range(_OUT_C // 16):
                    sl = pl.ds(c * 16, 16)
                    acc = scratch[0, sl]
                    for rr in range(1, _K):
                        acc = jnp.maximum(acc, scratch[rr, sl])
                    o_vmem[r, sl] = acc

        pltpu.emit_pipeline(
            body,
            grid=(m // _SC_ROWS,),
            in_specs=[pl.BlockSpec((_SC_ROWS, _K), lambda i: (i, 0))],
            out_specs=[pl.BlockSpec((_SC_ROWS, _OUT_C), lambda i: (i, 0))],
            core_axis_name=("core", "subcore"),
            dimension_semantics=(pltpu.PARALLEL,),
        )(i_hbm, o_hbm)

    return gather_kernel(z, idx)


def kernel(feats, xyz, offset, gamma, beta, W):
    px = xyz[:, 0].reshape(_B, _SEG)
    py = xyz[:, 1].reshape(_B, _SEG)
    pz = xyz[:, 2].reshape(_B, _SEG)

    q = _run_fps(px, py, pz)
    qx = q[:, 0:_B].T
    qy = q[:, _B:2 * _B].T
    qz = q[:, 2 * _B:3 * _B].T
    idx = _run_knn(qx, qy, qz, px, py, pz)
    z = _run_z(feats, gamma, beta, W.T)
    outp = _run_gather_max(z, idx.reshape(_B * _NSP, _K))

    out = outp.reshape(_B, _NSP, _OUT_C)[:, :_NS].reshape(_B * _NS, _OUT_C)
    n_xyz = jnp.stack([qx, qy, qz], axis=-1)[:, :_NS].reshape(_B * _NS, 3)

    diffs = jnp.diff(offset, prepend=jnp.zeros((1,), offset.dtype))
    per = (diffs.astype(jnp.float32) * 0.25).astype(jnp.int32) + 1
    n_offset = jnp.cumsum(per).astype(jnp.int32)
    return (out, n_xyz, n_offset)


# speedup vs baseline: 1.0408x; 1.0408x over previous
import jax
import jax.numpy as jnp
from jax import lax
from jax.experimental import pallas as pl
from jax.experimental.pallas import tpu as pltpu
from jax.experimental.pallas import tpu_sc as plsc

_B = 4
_SEG = 8192
_NS = 2049
_K = 16
_IN_C = 64
_OUT_C = 128
_QR = 17
_NSP = _QR * 128
_N = _B * _SEG
_SR = _SEG // 128


def _fps_body(x_ref, y_ref, z_ref, q_ref):
    shape = (_B, _SR, 128)
    lin = (lax.broadcasted_iota(jnp.int32, shape, 1) * 128
           + lax.broadcasted_iota(jnp.int32, shape, 2))
    q_ref[...] = jnp.zeros((_NSP, 16), jnp.float32)

    def store_q(i, qx, qy, qz):
        for s in range(_B):
            q_ref[pl.ds(i, 1), s:s + 1] = qx[s]
            q_ref[pl.ds(i, 1), _B + s:_B + s + 1] = qy[s]
            q_ref[pl.ds(i, 1), 2 * _B + s:2 * _B + s + 1] = qz[s]

    def extract(wm):
        qx = jnp.sum(jnp.where(wm, x_ref[...], 0.0), axis=(1, 2), keepdims=True)
        qy = jnp.sum(jnp.where(wm, y_ref[...], 0.0), axis=(1, 2), keepdims=True)
        qz = jnp.sum(jnp.where(wm, z_ref[...], 0.0), axis=(1, 2), keepdims=True)
        return qx, qy, qz

    def body(i, carry):
        dists, qx, qy, qz = carry
        dx = x_ref[...] - qx
        dy = y_ref[...] - qy
        dz = z_ref[...] - qz
        d = (dx * dx + dz * dz) + dy * dy
        dists = jnp.minimum(dists, d)
        mx = jnp.max(dists, axis=(1, 2), keepdims=True)
        cand = jnp.where(dists == mx, lin, _SEG)
        j = jnp.min(cand, axis=(1, 2), keepdims=True)
        qx2, qy2, qz2 = extract(cand == j)
        store_q(i + 1, qx2, qy2, qz2)
        return dists, qx2, qy2, qz2

    qx0, qy0, qz0 = extract(lin == 0)
    store_q(0, qx0, qy0, qz0)
    init = (jnp.full(shape, jnp.inf, jnp.float32), qx0, qy0, qz0)
    lax.fori_loop(0, _NS - 1, body, init)


def _run_fps(px, py, pz):
    return pl.pallas_call(
        _fps_body,
        out_shape=jax.ShapeDtypeStruct((_NSP, 16), jnp.float32),
    )(px.reshape(_B, _SR, 128), py.reshape(_B, _SR, 128),
      pz.reshape(_B, _SR, 128))


def _z_body(f_ref, g_ref, b_ref, w_ref, z_ref):
    x = f_ref[...]
    mu = jnp.mean(x, axis=1, keepdims=True)
    xc = x - mu
    var = jnp.mean(xc * xc, axis=1, keepdims=True)
    gn = xc / jnp.sqrt(var + 1e-5) * g_ref[...] + b_ref[...]
    z_ref[...] = jnp.dot(gn, w_ref[...],
                         preferred_element_type=jnp.float32,
                         precision=lax.Precision.HIGHEST)


def _run_z(feats, gamma, beta, wT):
    rows = min(2048, _N)
    return pl.pallas_call(
        _z_body,
        grid=(_N // rows,),
        in_specs=[
            pl.BlockSpec((rows, _IN_C), lambda i: (i, 0)),
            pl.BlockSpec((1, _IN_C), lambda i: (0, 0)),
            pl.BlockSpec((1, _IN_C), lambda i: (0, 0)),
            pl.BlockSpec((_IN_C, _OUT_C), lambda i: (0, 0)),
        ],
        out_specs=pl.BlockSpec((rows, _OUT_C), lambda i: (i, 0)),
        out_shape=jax.ShapeDtypeStruct((_N, _OUT_C), jnp.float32),
    )(feats, gamma.reshape(1, _IN_C), beta.reshape(1, _IN_C), wT)


def _knn_body(qx_ref, qy_ref, qz_ref, px_ref, py_ref, pz_ref, idx_ref, d_ref):
    s = pl.program_id(0)
    qx = jnp.swapaxes(qx_ref[0, 0], 0, 1)
    qy = jnp.swapaxes(qy_ref[0, 0], 0, 1)
    qz = jnp.swapaxes(qz_ref[0, 0], 0, 1)
    dx = qx - px_ref[0]
    dy = qy - py_ref[0]
    dz = qz - pz_ref[0]
    d_ref[...] = dx * dx + dy * dy + dz * dz
    iota = lax.broadcasted_iota(jnp.int32, (128, _SEG), 1)
    for k in range(_K):
        dmat = d_ref[...]
        j = jnp.argmin(dmat, axis=1).astype(jnp.int32)[:, None]
        idx_ref[0, :, k:k + 1] = j + s * _SEG
        d_ref[...] = jnp.where(iota == j, jnp.inf, dmat)


def _run_knn(qx, qy, qz, px, py, pz):
    qspec = pl.BlockSpec((1, 1, 1, 128), lambda s, qb: (s, qb, 0, 0))
    pspec = pl.BlockSpec((1, 1, _SEG), lambda s, qb: (s, 0, 0))
    return pl.pallas_call(
        _knn_body,
        grid=(_B, _QR),
        in_specs=[qspec, qspec, qspec, pspec, pspec, pspec],
        out_specs=pl.BlockSpec((1, 128, _K), lambda s, qb: (s, qb, 0)),
        out_shape=jax.ShapeDtypeStruct((_B, _NSP, _K), jnp.int32),
        scratch_shapes=[pltpu.VMEM((128, _SEG), jnp.float32)],
    )(qx.reshape(_B, _QR, 1, 128), qy.reshape(_B, _QR, 1, 128),
      qz.reshape(_B, _QR, 1, 128),
      px.reshape(_B, 1, _SEG), py.reshape(_B, 1, _SEG),
      pz.reshape(_B, 1, _SEG))


_SC_ROWS = 8


def _run_gather_max(z, idx):
    m = _B * _NSP
    mesh = plsc.VectorSubcoreMesh(core_axis_name="core",
                                  subcore_axis_name="subcore")

    @pl.kernel(out_type=jax.ShapeDtypeStruct((m, _OUT_C), jnp.float32),
               mesh=mesh,
               scratch_types=[pltpu.VMEM((_K, _OUT_C), jnp.float32)])
    def gather_kernel(z_hbm, i_hbm, o_hbm, scratch):
        def body(i_vmem, o_vmem):
            @pl.loop(0, _SC_ROWS)
            def _(r):
                pltpu.sync_copy(z_hbm.at[i_vmem.at[r]], scratch)
                for c in range(_OUT_C // 16):
                    sl = pl.ds(c * 16, 16)
                    acc = scratch[0, sl]
                    for rr in range(1, _K):
                        acc = jnp.maximum(acc, scratch[rr, sl])
                    o_vmem[r, sl] = acc

        pltpu.emit_pipeline(
            body,
            grid=(m // _SC_ROWS,),
            in_specs=[pl.BlockSpec((_SC_ROWS, _K), lambda i: (i, 0))],
            out_specs=[pl.BlockSpec((_SC_ROWS, _OUT_C), lambda i: (i, 0))],
            core_axis_name=("core", "subcore"),
            dimension_semantics=(pltpu.PARALLEL,),
        )(i_hbm, o_hbm)

    return gather_kernel(z, idx)


def kernel(feats, xyz, offset, gamma, beta, W):
    px = xyz[:, 0].reshape(_B, _SEG)
    py = xyz[:, 1].reshape(_B, _SEG)
    pz = xyz[:, 2].reshape(_B, _SEG)

    q = _run_fps(px, py, pz)
    qx = q[:, 0:_B].T
    qy = q[:, _B:2 * _B].T
    qz = q[:, 2 * _B:3 * _B].T
    idx = _run_knn(qx, qy, qz, px, py, pz)
    z = _run_z(feats, gamma, beta, W.T)
    outp = _run_gather_max(z, idx.reshape(_B * _NSP, _K))

    out = outp.reshape(_B, _NSP, _OUT_C)[:, :_NS].reshape(_B * _NS, _OUT_C)
    n_xyz = jnp.stack([qx, qy, qz], axis=-1)[:, :_NS].reshape(_B * _NS, 3)

    diffs = jnp.diff(offset, prepend=jnp.zeros((1,), offset.dtype))
    per = (diffs.astype(jnp.float32) * 0.25).astype(jnp.int32) + 1
    n_offset = jnp.cumsum(per).astype(jnp.int32)
    return (out, n_xyz, n_offset)


# speedup vs baseline: 1.0945x; 1.0516x over previous
import jax
import jax.numpy as jnp
from jax import lax
from jax.experimental import pallas as pl
from jax.experimental.pallas import tpu as pltpu
from jax.experimental.pallas import tpu_sc as plsc

_B = 4
_SEG = 8192
_NS = 2049
_K = 16
_IN_C = 64
_OUT_C = 128
_QR = 17
_NSP = _QR * 128
_N = _B * _SEG
_SR = _SEG // 128


def _fps_body(x_ref, y_ref, z_ref, q_ref):
    shape = (_B, _SR, 128)
    lin = (lax.broadcasted_iota(jnp.int32, shape, 1) * 128
           + lax.broadcasted_iota(jnp.int32, shape, 2))
    q_ref[...] = jnp.zeros((_NSP, 16), jnp.float32)

    def store_q(i, qx, qy, qz):
        for s in range(_B):
            q_ref[pl.ds(i, 1), s:s + 1] = qx[s]
            q_ref[pl.ds(i, 1), _B + s:_B + s + 1] = qy[s]
            q_ref[pl.ds(i, 1), 2 * _B + s:2 * _B + s + 1] = qz[s]

    def extract(wm):
        qx = jnp.sum(jnp.where(wm, x_ref[...], 0.0), axis=(1, 2), keepdims=True)
        qy = jnp.sum(jnp.where(wm, y_ref[...], 0.0), axis=(1, 2), keepdims=True)
        qz = jnp.sum(jnp.where(wm, z_ref[...], 0.0), axis=(1, 2), keepdims=True)
        return qx, qy, qz

    def body(i, carry):
        dists, qx, qy, qz = carry
        dx = x_ref[...] - qx
        dy = y_ref[...] - qy
        dz = z_ref[...] - qz
        d = (dx * dx + dz * dz) + dy * dy
        dists = jnp.minimum(dists, d)
        mx = jnp.max(dists, axis=(1, 2), keepdims=True)
        cand = jnp.where(dists == mx, lin, _SEG)
        j = jnp.min(cand, axis=(1, 2), keepdims=True)
        qx2, qy2, qz2 = extract(cand == j)
        store_q(i + 1, qx2, qy2, qz2)
        return dists, qx2, qy2, qz2

    qx0, qy0, qz0 = extract(lin == 0)
    store_q(0, qx0, qy0, qz0)
    init = (jnp.full(shape, jnp.inf, jnp.float32), qx0, qy0, qz0)
    lax.fori_loop(0, _NS - 1, body, init)


def _run_fps(px, py, pz):
    return pl.pallas_call(
        _fps_body,
        out_shape=jax.ShapeDtypeStruct((_NSP, 16), jnp.float32),
    )(px.reshape(_B, _SR, 128), py.reshape(_B, _SR, 128),
      pz.reshape(_B, _SR, 128))


def _z_body(f_ref, g_ref, b_ref, w_ref, z_ref):
    x = f_ref[...]
    mu = jnp.mean(x, axis=1, keepdims=True)
    xc = x - mu
    var = jnp.mean(xc * xc, axis=1, keepdims=True)
    gn = xc / jnp.sqrt(var + 1e-5) * g_ref[...] + b_ref[...]
    z_ref[...] = jnp.dot(gn, w_ref[...],
                         preferred_element_type=jnp.float32,
                         precision=lax.Precision.HIGHEST)


def _run_z(feats, gamma, beta, wT):
    rows = min(2048, _N)
    return pl.pallas_call(
        _z_body,
        grid=(_N // rows,),
        in_specs=[
            pl.BlockSpec((rows, _IN_C), lambda i: (i, 0)),
            pl.BlockSpec((1, _IN_C), lambda i: (0, 0)),
            pl.BlockSpec((1, _IN_C), lambda i: (0, 0)),
            pl.BlockSpec((_IN_C, _OUT_C), lambda i: (0, 0)),
        ],
        out_specs=pl.BlockSpec((rows, _OUT_C), lambda i: (i, 0)),
        out_shape=jax.ShapeDtypeStruct((_N, _OUT_C), jnp.float32),
    )(feats, gamma.reshape(1, _IN_C), beta.reshape(1, _IN_C), wT)


def _make_knn_body(base):
    def _knn_body(qx_ref, qy_ref, qz_ref, px_ref, py_ref, pz_ref, idx_ref,
                  d_ref):
        qx = jnp.swapaxes(qx_ref[0], 0, 1)
        qy = jnp.swapaxes(qy_ref[0], 0, 1)
        qz = jnp.swapaxes(qz_ref[0], 0, 1)
        dx = qx - px_ref[...]
        dy = qy - py_ref[...]
        dz = qz - pz_ref[...]
        d_ref[...] = dx * dx + dy * dy + dz * dz
        iota = lax.broadcasted_iota(jnp.int32, (128, _SEG), 1)
        for k in range(_K):
            dmat = d_ref[...]
            j = jnp.argmin(dmat, axis=1).astype(jnp.int32)[:, None]
            idx_ref[:, k:k + 1] = j + base
            d_ref[...] = jnp.where(iota == j, jnp.inf, dmat)
    return _knn_body


def _run_knn_seg(s, qx, qy, qz, px, py, pz):
    qspec = pl.BlockSpec((1, 1, 128), lambda qb: (qb, 0, 0))
    pspec = pl.BlockSpec((1, _SEG), lambda qb: (0, 0))
    return pl.pallas_call(
        _make_knn_body(s * _SEG),
        grid=(_QR,),
        in_specs=[qspec, qspec, qspec, pspec, pspec, pspec],
        out_specs=pl.BlockSpec((128, _K), lambda qb: (qb, 0)),
        out_shape=jax.ShapeDtypeStruct((_NSP, _K), jnp.int32),
        scratch_shapes=[pltpu.VMEM((128, _SEG), jnp.float32)],
    )(qx.reshape(_QR, 1, 128), qy.reshape(_QR, 1, 128),
      qz.reshape(_QR, 1, 128),
      px.reshape(1, _SEG), py.reshape(1, _SEG), pz.reshape(1, _SEG))


_SC_ROWS = 8


def _run_gather_max(z, idx):
    m = idx.shape[0]
    mesh = plsc.VectorSubcoreMesh(core_axis_name="core",
                                  subcore_axis_name="subcore")

    @pl.kernel(out_type=jax.ShapeDtypeStruct((m, _OUT_C), jnp.float32),
               mesh=mesh,
               scratch_types=[pltpu.VMEM((_K, _OUT_C), jnp.float32)])
    def gather_kernel(z_hbm, i_hbm, o_hbm, scratch):
        def body(i_vmem, o_vmem):
            @pl.loop(0, _SC_ROWS)
            def _(r):
                pltpu.sync_copy(z_hbm.at[i_vmem.at[r]], scratch)
                for c in range(_OUT_C // 16):
                    sl = pl.ds(c * 16, 16)
                    acc = scratch[0, sl]
                    for rr in range(1, _K):
                        acc = jnp.maximum(acc, scratch[rr, sl])
                    o_vmem[r, sl] = acc

        pltpu.emit_pipeline(
            body,
            grid=(m // _SC_ROWS,),
            in_specs=[pl.BlockSpec((_SC_ROWS, _K), lambda i: (i, 0))],
            out_specs=[pl.BlockSpec((_SC_ROWS, _OUT_C), lambda i: (i, 0))],
            core_axis_name=("core", "subcore"),
            dimension_semantics=(pltpu.PARALLEL,),
        )(i_hbm, o_hbm)

    return gather_kernel(z, idx)


def kernel(feats, xyz, offset, gamma, beta, W):
    px = xyz[:, 0].reshape(_B, _SEG)
    py = xyz[:, 1].reshape(_B, _SEG)
    pz = xyz[:, 2].reshape(_B, _SEG)

    q = _run_fps(px, py, pz)
    qx = q[:, 0:_B].T
    qy = q[:, _B:2 * _B].T
    qz = q[:, 2 * _B:3 * _B].T
    z = _run_z(feats, gamma, beta, W.T)
    outs = []
    for s in range(_B):
        idx_s = _run_knn_seg(s, qx[s], qy[s], qz[s], px[s], py[s], pz[s])
        outs.append(_run_gather_max(z, idx_s))
    outp = jnp.stack(outs)

    out = outp[:, :_NS].reshape(_B * _NS, _OUT_C)
    n_xyz = jnp.stack([qx, qy, qz], axis=-1)[:, :_NS].reshape(_B * _NS, 3)

    diffs = jnp.diff(offset, prepend=jnp.zeros((1,), offset.dtype))
    per = (diffs.astype(jnp.float32) * 0.25).astype(jnp.int32) + 1
    n_offset = jnp.cumsum(per).astype(jnp.int32)
    return (out, n_xyz, n_offset)
